# baseline (speedup 1.0000x reference)
import jax
import jax.numpy as jnp
from jax.experimental import pallas as pl


def _body(a_ref, b_ref, c_ref, d_ref, o_ref):
    o_ref[:] = jnp.sum(a_ref[:] * b_ref[:] + c_ref[:] * d_ref[:], axis=1)


def kernel(gu, gi, gut, git):
    B, D = gu.shape
    BLK = 2048
    return pl.pallas_call(
        _body,
        grid=(B // BLK,),
        in_specs=[pl.BlockSpec((BLK, D), lambda i: (i, 0))] * 4,
        out_specs=pl.BlockSpec((BLK,), lambda i: (i,)),
        out_shape=jax.ShapeDtypeStruct((B,), jnp.float32),
    )(gu, gi, gut, git)
